# Initial kernel scaffold; baseline (speedup 1.0000x reference)
import jax
import jax.numpy as jnp
from jax.experimental import pallas as pl


def kernel(word_table, conn_NOUN, conn_VERB, conn_ADJ, conn_ADV, text, topic, idx_NOUN, idx_VERB, idx_ADJ, idx_ADV, txt_l, top_l):
    raise NotImplementedError("write your pallas kernel here")


# speedup vs baseline: 6.5677x; 6.5677x over previous
import functools
import jax
import jax.numpy as jnp
from jax import lax
from jax.experimental import pallas as pl
from jax.experimental.pallas import tpu as pltpu
from jax.experimental.pallas import tpu_sc as plsc

_VOCAB = 100000
_DIM = 64
_CDIM = 16
_B = 1024
_L = 200
_LT = 20

_NC = 2
_NS = 16
_NW = _NC * _NS

_TXT = _B * _L
_TOP = _B * _LT
_CHUNK = 128
_TXT_PW = _TXT // _NW
_TOP_PW = _TOP // _NW
_TXT_CH = _TXT_PW // _CHUNK
_TOP_CH = _TOP_PW // _CHUNK


def _sc_body(word_hbm, c0_hbm, c1_hbm, c2_hbm, c3_hbm,
             text_hbm, topic_hbm, i0_hbm, i1_hbm, i2_hbm, i3_hbm,
             word_out, top_out, conn_out, pos_out,
             idx_txt, idx_top, idx_pos, wbuf,
             p0, p1, p2, p3, cb0, cb1, cb2, cb3, sem):
  w = lax.axis_index("s") * _NC + lax.axis_index("c")
  txt_base = w * _TXT_PW
  top_base = w * _TOP_PW

  pltpu.sync_copy(text_hbm.at[w], idx_txt)
  pltpu.sync_copy(topic_hbm.at[w], idx_top)
  pltpu.sync_copy(i0_hbm.at[w], idx_pos.at[0])
  pltpu.sync_copy(i1_hbm.at[w], idx_pos.at[1])
  pltpu.sync_copy(i2_hbm.at[w], idx_pos.at[2])
  pltpu.sync_copy(i3_hbm.at[w], idx_pos.at[3])

  def txt_chunk(j, _):
    pltpu.async_copy(word_hbm.at[idx_txt.at[j]], wbuf, sem).wait()
    pltpu.sync_copy(wbuf, word_out.at[pl.ds(txt_base + j * _CHUNK, _CHUNK)])
    return _
  lax.fori_loop(0, _TXT_CH, txt_chunk, None)

  def top_chunk(j, _):
    pltpu.async_copy(word_hbm.at[idx_top.at[j]], wbuf, sem).wait()
    pltpu.sync_copy(wbuf, top_out.at[pl.ds(top_base + j * _CHUNK, _CHUNK)])
    return _
  lax.fori_loop(0, _TOP_CH, top_chunk, None)

  def pos_chunk(j, _):
    g0 = pltpu.async_copy(word_hbm.at[idx_pos.at[0, j]], p0, sem)
    g1 = pltpu.async_copy(word_hbm.at[idx_pos.at[1, j]], p1, sem)
    g2 = pltpu.async_copy(word_hbm.at[idx_pos.at[2, j]], p2, sem)
    g3 = pltpu.async_copy(word_hbm.at[idx_pos.at[3, j]], p3, sem)
    h0 = pltpu.async_copy(c0_hbm.at[idx_pos.at[0, j]], cb0, sem)
    h1 = pltpu.async_copy(c1_hbm.at[idx_pos.at[1, j]], cb1, sem)
    h2 = pltpu.async_copy(c2_hbm.at[idx_pos.at[2, j]], cb2, sem)
    h3 = pltpu.async_copy(c3_hbm.at[idx_pos.at[3, j]], cb3, sem)
    g0.wait(); g1.wait(); g2.wait(); g3.wait()
    h0.wait(); h1.wait(); h2.wait(); h3.wait()

    def row_add(r, _):
      for c in range(_DIM // 16):
        s = pl.ds(c * 16, 16)
        p0[r, s] = p0[r, s] + p1[r, s] + p2[r, s] + p3[r, s]
      cb0[r, :] = cb0[r, :] + cb1[r, :] + cb2[r, :] + cb3[r, :]
      return _
    lax.fori_loop(0, _CHUNK, row_add, None)

    dst = pl.ds(txt_base + j * _CHUNK, _CHUNK)
    pltpu.sync_copy(p0, pos_out.at[dst])
    pltpu.sync_copy(cb0, conn_out.at[dst])
    return _
  lax.fori_loop(0, _TXT_CH, pos_chunk, None)


@jax.jit
def _run(word_table, c0, c1, c2, c3, text, topic, i0, i1, i2, i3):
  mesh = plsc.VectorSubcoreMesh(core_axis_name="c", subcore_axis_name="s")
  f = pl.kernel(
      _sc_body,
      out_type=[
          jax.ShapeDtypeStruct((_TXT, _DIM), jnp.float32),
          jax.ShapeDtypeStruct((_TOP, _DIM), jnp.float32),
          jax.ShapeDtypeStruct((_TXT, _CDIM), jnp.float32),
          jax.ShapeDtypeStruct((_TXT, _DIM), jnp.float32),
      ],
      mesh=mesh,
      compiler_params=pltpu.CompilerParams(use_tc_tiling_on_sc=False),
      scratch_types=[
          pltpu.VMEM((_TXT_CH, _CHUNK), jnp.int32),
          pltpu.VMEM((_TOP_CH, _CHUNK), jnp.int32),
          pltpu.VMEM((4, _TXT_CH, _CHUNK), jnp.int32),
          pltpu.VMEM((_CHUNK, _DIM), jnp.float32),
          pltpu.VMEM((_CHUNK, _DIM), jnp.float32),
          pltpu.VMEM((_CHUNK, _DIM), jnp.float32),
          pltpu.VMEM((_CHUNK, _DIM), jnp.float32),
          pltpu.VMEM((_CHUNK, _DIM), jnp.float32),
          pltpu.VMEM((_CHUNK, _CDIM), jnp.float32),
          pltpu.VMEM((_CHUNK, _CDIM), jnp.float32),
          pltpu.VMEM((_CHUNK, _CDIM), jnp.float32),
          pltpu.VMEM((_CHUNK, _CDIM), jnp.float32),
          pltpu.SemaphoreType.DMA,
      ],
  )
  return f(word_table, c0, c1, c2, c3, text, topic, i0, i1, i2, i3)


def kernel(word_table, conn_NOUN, conn_VERB, conn_ADJ, conn_ADV,
           text, topic, idx_NOUN, idx_VERB, idx_ADJ, idx_ADV,
           txt_l, top_l):
  text_r = text.reshape(_NW, _TXT_CH, _CHUNK)
  topic_r = topic.reshape(_NW, _TOP_CH, _CHUNK)
  i0 = idx_NOUN.reshape(_NW, _TXT_CH, _CHUNK)
  i1 = idx_VERB.reshape(_NW, _TXT_CH, _CHUNK)
  i2 = idx_ADJ.reshape(_NW, _TXT_CH, _CHUNK)
  i3 = idx_ADV.reshape(_NW, _TXT_CH, _CHUNK)
  word_E, top_E, conn_E, pos_E = _run(
      word_table, conn_NOUN, conn_VERB, conn_ADJ, conn_ADV,
      text_r, topic_r, i0, i1, i2, i3)
  return (word_E.reshape(_B, _L, _DIM),
          top_E.reshape(_B, _LT, _DIM),
          conn_E.reshape(_B, _L, _CDIM),
          pos_E.reshape(_B, _L, _DIM),
          txt_l, top_l)


# speedup vs baseline: 8.0720x; 1.2290x over previous
import jax
import jax.numpy as jnp
from jax import lax
from jax.experimental import pallas as pl
from jax.experimental.pallas import tpu as pltpu
from jax.experimental.pallas import tpu_sc as plsc

_VOCAB = 100000
_DIM = 64
_CDIM = 16
_B = 1024
_L = 200
_LT = 20

_NC = 2
_NS = 16
_NW = _NC * _NS

_TXT = _B * _L
_TOP = _B * _LT
_WRD = _TXT + _TOP
_CHUNK = 128
_WRD_PW = _WRD // _NW
_POS_PW = _TXT // _NW
_WRD_CH = _WRD_PW // _CHUNK
_POS_CH = _POS_PW // _CHUNK


def _sc_body(word_hbm, c0_hbm, c1_hbm, c2_hbm, c3_hbm,
             wrd_hbm, i0_hbm, i1_hbm, i2_hbm, i3_hbm,
             wrd_out, conn_out, pos_out,
             idx_wrd, idx_pos,
             wb0, wb1, p0, p1, cb0, cb1,
             wg0, wg1, ws0, ws1,
             pg0, pg1, pa0, pa1, pso0, pso1, cso0, cso1):
  w = lax.axis_index("s") * _NC + lax.axis_index("c")
  wrd_base = w * _WRD_PW
  pos_base = w * _POS_PW

  wb = [wb0, wb1]
  pb = [p0, p1]
  cb = [cb0, cb1]
  wg = [wg0, wg1]
  ws = [ws0, ws1]
  pg = [pg0, pg1]
  pa = [pa0, pa1]
  pso = [pso0, pso1]
  cso = [cso0, cso1]

  pltpu.sync_copy(wrd_hbm.at[w], idx_wrd)
  pltpu.sync_copy(i0_hbm.at[w], idx_pos.at[0])
  pltpu.sync_copy(i1_hbm.at[w], idx_pos.at[1])
  pltpu.sync_copy(i2_hbm.at[w], idx_pos.at[2])
  pltpu.sync_copy(i3_hbm.at[w], idx_pos.at[3])

  pltpu.async_copy(word_hbm.at[idx_wrd.at[0]], wb[0], wg[0])

  def wrd_pair(t, _):
    for s in range(2):
      j = 2 * t + s
      dst = wrd_out.at[pl.ds(wrd_base + j * _CHUNK, _CHUNK)]
      pltpu.make_async_copy(word_hbm.at[idx_wrd.at[j]], wb[s], wg[s]).wait()
      if s == 1:
        pltpu.make_async_copy(
            wb[0], wrd_out.at[pl.ds(wrd_base + j * _CHUNK - _CHUNK, _CHUNK)],
            ws[0]).wait()
      else:
        @pl.when(t > 0)
        def _():
          pltpu.make_async_copy(
              wb[1], wrd_out.at[pl.ds(wrd_base + j * _CHUNK - _CHUNK, _CHUNK)],
              ws[1]).wait()

      @pl.when(j + 1 < _WRD_CH)
      def _():
        pltpu.async_copy(word_hbm.at[idx_wrd.at[j + 1]], wb[1 - s], wg[1 - s])
      pltpu.async_copy(wb[s], dst, ws[s])
    return _

  lax.fori_loop(0, _WRD_CH // 2, wrd_pair, None)
  jlast = _WRD_CH - 1
  pltpu.make_async_copy(word_hbm.at[idx_wrd.at[jlast]], wb[0], wg[0]).wait()
  pltpu.make_async_copy(
      wb[1], wrd_out.at[pl.ds(wrd_base + (jlast - 1) * _CHUNK, _CHUNK)],
      ws[1]).wait()
  pltpu.async_copy(
      wb[0], wrd_out.at[pl.ds(wrd_base + jlast * _CHUNK, _CHUNK)], ws[0])

  pltpu.async_copy(word_hbm.at[idx_pos.at[0, 0]], pb[0], pg[0])
  pltpu.async_copy(c0_hbm.at[idx_pos.at[0, 0]], cb[0], pg[0])

  def pos_pair(t, _):
    for s in range(2):
      j = 2 * t + s
      dst = pl.ds(pos_base + j * _CHUNK, _CHUNK)
      pdst = pl.ds(pos_base + j * _CHUNK - _CHUNK, _CHUNK)
      pltpu.make_async_copy(word_hbm.at[idx_pos.at[0, j]], pb[s], pg[s]).wait()
      pltpu.make_async_copy(c0_hbm.at[idx_pos.at[0, j]], cb[s], pg[s]).wait()
      pltpu.async_copy(word_hbm.at[idx_pos.at[1, j]], pb[s], pa[s], add=True)
      pltpu.async_copy(word_hbm.at[idx_pos.at[2, j]], pb[s], pa[s], add=True)
      pltpu.async_copy(word_hbm.at[idx_pos.at[3, j]], pb[s], pa[s], add=True)
      pltpu.async_copy(c1_hbm.at[idx_pos.at[1, j]], cb[s], pa[s], add=True)
      pltpu.async_copy(c2_hbm.at[idx_pos.at[2, j]], cb[s], pa[s], add=True)
      pltpu.async_copy(c3_hbm.at[idx_pos.at[3, j]], cb[s], pa[s], add=True)
      o = 1 - s
      if s == 1:
        pltpu.make_async_copy(pb[0], pos_out.at[pdst], pso[0]).wait()
        pltpu.make_async_copy(cb[0], conn_out.at[pdst], cso[0]).wait()
      else:
        @pl.when(t > 0)
        def _():
          pltpu.make_async_copy(pb[1], pos_out.at[pdst], pso[1]).wait()
          pltpu.make_async_copy(cb[1], conn_out.at[pdst], cso[1]).wait()

      @pl.when(j + 1 < _POS_CH)
      def _():
        pltpu.async_copy(word_hbm.at[idx_pos.at[0, j + 1]], pb[o], pg[o])
        pltpu.async_copy(c0_hbm.at[idx_pos.at[0, j + 1]], cb[o], pg[o])
      pltpu.make_async_copy(word_hbm.at[idx_pos.at[1, j]], pb[s], pa[s]).wait()
      pltpu.make_async_copy(word_hbm.at[idx_pos.at[2, j]], pb[s], pa[s]).wait()
      pltpu.make_async_copy(word_hbm.at[idx_pos.at[3, j]], pb[s], pa[s]).wait()
      pltpu.make_async_copy(c1_hbm.at[idx_pos.at[1, j]], cb[s], pa[s]).wait()
      pltpu.make_async_copy(c2_hbm.at[idx_pos.at[2, j]], cb[s], pa[s]).wait()
      pltpu.make_async_copy(c3_hbm.at[idx_pos.at[3, j]], cb[s], pa[s]).wait()
      pltpu.async_copy(pb[s], pos_out.at[dst], pso[s])
      pltpu.async_copy(cb[s], conn_out.at[dst], cso[s])
    return _

  lax.fori_loop(0, _POS_CH // 2, pos_pair, None)
  end1 = pl.ds(pos_base + (_POS_CH - 1) * _CHUNK, _CHUNK)
  pltpu.make_async_copy(pb[1], pos_out.at[end1], pso[1]).wait()
  pltpu.make_async_copy(cb[1], conn_out.at[end1], cso[1]).wait()
  pltpu.make_async_copy(
      wb[0], wrd_out.at[pl.ds(wrd_base + jlast * _CHUNK, _CHUNK)], ws[0]).wait()


@jax.jit
def _run(word_table, c0, c1, c2, c3, wrd_idx, i0, i1, i2, i3):
  mesh = plsc.VectorSubcoreMesh(core_axis_name="c", subcore_axis_name="s")
  f = pl.kernel(
      _sc_body,
      out_type=[
          jax.ShapeDtypeStruct((_WRD, _DIM), jnp.float32),
          jax.ShapeDtypeStruct((_TXT, _CDIM), jnp.float32),
          jax.ShapeDtypeStruct((_TXT, _DIM), jnp.float32),
      ],
      mesh=mesh,
      compiler_params=pltpu.CompilerParams(use_tc_tiling_on_sc=False),
      scratch_types=[
          pltpu.VMEM((_WRD_CH, _CHUNK), jnp.int32),
          pltpu.VMEM((4, _POS_CH, _CHUNK), jnp.int32),
          pltpu.VMEM((_CHUNK, _DIM), jnp.float32),
          pltpu.VMEM((_CHUNK, _DIM), jnp.float32),
          pltpu.VMEM((_CHUNK, _DIM), jnp.float32),
          pltpu.VMEM((_CHUNK, _DIM), jnp.float32),
          pltpu.VMEM((_CHUNK, _CDIM), jnp.float32),
          pltpu.VMEM((_CHUNK, _CDIM), jnp.float32),
      ] + [pltpu.SemaphoreType.DMA] * 12,
  )
  return f(word_table, c0, c1, c2, c3, wrd_idx, i0, i1, i2, i3)


def kernel(word_table, conn_NOUN, conn_VERB, conn_ADJ, conn_ADV,
           text, topic, idx_NOUN, idx_VERB, idx_ADJ, idx_ADV,
           txt_l, top_l):
  wrd_idx = jnp.concatenate([text.reshape(-1), topic.reshape(-1)])
  wrd_idx = wrd_idx.reshape(_NW, _WRD_CH, _CHUNK)
  i0 = idx_NOUN.reshape(_NW, _POS_CH, _CHUNK)
  i1 = idx_VERB.reshape(_NW, _POS_CH, _CHUNK)
  i2 = idx_ADJ.reshape(_NW, _POS_CH, _CHUNK)
  i3 = idx_ADV.reshape(_NW, _POS_CH, _CHUNK)
  wrd_E, conn_E, pos_E = _run(
      word_table, conn_NOUN, conn_VERB, conn_ADJ, conn_ADV,
      wrd_idx, i0, i1, i2, i3)
  word_E = wrd_E[:_TXT].reshape(_B, _L, _DIM)
  top_E = wrd_E[_TXT:].reshape(_B, _LT, _DIM)
  return (word_E, top_E,
          conn_E.reshape(_B, _L, _CDIM),
          pos_E.reshape(_B, _L, _DIM),
          txt_l, top_l)
